# Initial kernel scaffold; baseline (speedup 1.0000x reference)
import functools

import jax
import jax.numpy as jnp
from jax import lax
from jax.experimental import pallas as pl
from jax.experimental.pallas import tpu as pltpu
from jax.experimental.pallas import tpu_sc as plsc

N = 100000
E = 3200000
NC, NS = 2, 16
LANES = 128
E_PAD = 3276800
ROWS = E_PAD // LANES
RPT = 6256
N_PAD = NS * RPT
SENT = N_PAD - 1
CHUNK = 16
BLK = 2000
GRID = N // BLK

_mesh = plsc.VectorSubcoreMesh(core_axis_name="c", subcore_axis_name="s")


def _deg_body(dst_rows, ones_hbm, zeros_hbm, out_hbm, idx_v, ones_v, deg_sh):
    cid = lax.axis_index("c")
    sid = lax.axis_index("s")
    pltpu.sync_copy(ones_hbm, ones_v)
    pltpu.sync_copy(zeros_hbm.at[pl.ds(sid * RPT, RPT)],
                    deg_sh.at[pl.ds(sid * RPT, RPT)])
    plsc.subcore_barrier()
    rows_per_tile = ROWS // (NC * NS)
    base = (cid * NS + sid) * rows_per_tile

    def chunk(i, carry):
        r0 = base + i * CHUNK
        pltpu.sync_copy(dst_rows.at[pl.ds(r0, CHUNK)], idx_v)
        for j in range(CHUNK):
            pltpu.sync_copy(ones_v, deg_sh.at[idx_v.at[j]], add=True)
        return carry

    lax.fori_loop(0, rows_per_tile // CHUNK, chunk, 0)
    plsc.subcore_barrier()
    pltpu.sync_copy(deg_sh.at[pl.ds(sid * RPT, RPT)],
                    out_hbm.at[cid, pl.ds(sid * RPT, RPT)])


_deg_call = pl.kernel(
    _deg_body,
    out_type=jax.ShapeDtypeStruct((NC, N_PAD), jnp.float32),
    mesh=_mesh,
    scratch_types=[
        pltpu.VMEM((CHUNK, LANES), jnp.int32),
        pltpu.VMEM((LANES,), jnp.float32),
        pltpu.VMEM_SHARED((N_PAD,), jnp.float32),
    ],
)


def _edge_pass_body(dual, *refs):
    if dual:
        (t0, t1, src_rows, dst_rows, zeros_hbm, out_hbm,
         sidx, didx, rows_v, acc_sh, sem) = refs
    else:
        (t0, src_rows, dst_rows, zeros_hbm, out_hbm,
         sidx, didx, rows_v, acc_sh, sem) = refs
        t1 = t0
    cid = lax.axis_index("c")
    sid = lax.axis_index("s")
    pltpu.sync_copy(zeros_hbm.at[pl.ds(sid * RPT, RPT)],
                    acc_sh.at[pl.ds(sid * RPT, RPT)])
    plsc.subcore_barrier()
    if dual:
        rows_per_tile = ROWS // NS
        base = sid * rows_per_tile
    else:
        rows_per_tile = ROWS // (NC * NS)
        base = (cid * NS + sid) * rows_per_tile

    def chunk(i, carry):
        r0 = base + i * CHUNK
        pltpu.sync_copy(src_rows.at[pl.ds(r0, CHUNK)], sidx)
        pltpu.sync_copy(dst_rows.at[pl.ds(r0, CHUNK)], didx)

        def gather(tbl):
            copies = [
                pltpu.async_copy(tbl.at[sidx.at[j]],
                                 rows_v.at[pl.ds(j * LANES, LANES)], sem)
                for j in range(CHUNK)
            ]
            for c in copies:
                c.wait()

        if dual:
            @pl.when(cid == 0)
            def _():
                gather(t0)

            @pl.when(cid == 1)
            def _():
                gather(t1)
        else:
            gather(t0)
        for j in range(CHUNK):
            pltpu.sync_copy(rows_v.at[pl.ds(j * LANES, LANES)],
                            acc_sh.at[didx.at[j]], add=True)
        return carry

    lax.fori_loop(0, rows_per_tile // CHUNK, chunk, 0)
    plsc.subcore_barrier()
    pltpu.sync_copy(acc_sh.at[pl.ds(sid * RPT, RPT)],
                    out_hbm.at[cid, pl.ds(sid * RPT, RPT)])


def _make_edge_pass(dual):
    return pl.kernel(
        functools.partial(_edge_pass_body, dual),
        out_type=jax.ShapeDtypeStruct((NC, N_PAD, 16), jnp.float32),
        mesh=_mesh,
        scratch_types=[
            pltpu.VMEM((CHUNK, LANES), jnp.int32),
            pltpu.VMEM((CHUNK, LANES), jnp.int32),
            pltpu.VMEM((CHUNK * LANES, 16), jnp.float32),
            pltpu.VMEM_SHARED((N_PAD, 16), jnp.float32),
            pltpu.SemaphoreType.DMA,
        ],
    )


_pass_b = _make_edge_pass(True)
_pass_c = _make_edge_pass(False)


def _prep1_body(x_ref, w_ref, dega_ref, degb_ref, h_ref, g0_ref, g1_ref):
    dis = lax.rsqrt(dega_ref[...] + degb_ref[...] + 1.0)
    h = jnp.dot(x_ref[...], w_ref[...], preferred_element_type=jnp.float32)
    h_ref[...] = h
    g = h * dis
    g0_ref[...] = g[:, :16]
    g1_ref[...] = g[:, 16:]


_prep1 = pl.pallas_call(
    _prep1_body,
    grid=(GRID,),
    in_specs=[
        pl.BlockSpec((BLK, 16), lambda i: (i, 0)),
        pl.BlockSpec((16, 32), lambda i: (0, 0)),
        pl.BlockSpec((BLK, 1), lambda i: (i, 0)),
        pl.BlockSpec((BLK, 1), lambda i: (i, 0)),
    ],
    out_specs=[
        pl.BlockSpec((BLK, 32), lambda i: (i, 0)),
        pl.BlockSpec((BLK, 16), lambda i: (i, 0)),
        pl.BlockSpec((BLK, 16), lambda i: (i, 0)),
    ],
    out_shape=[
        jax.ShapeDtypeStruct((N_PAD, 32), jnp.float32),
        jax.ShapeDtypeStruct((N_PAD, 16), jnp.float32),
        jax.ShapeDtypeStruct((N_PAD, 16), jnp.float32),
    ],
)


def _mid_body(acc_ref, h1_ref, dega_ref, degb_ref, b1_ref, w2_ref,
              h2_ref, g2_ref):
    dis = lax.rsqrt(dega_ref[...] + degb_ref[...] + 1.0)
    accc = jnp.concatenate([acc_ref[0], acc_ref[1]], axis=1)
    out1 = jnp.maximum(
        dis * accc + (dis * dis) * h1_ref[...] + b1_ref[...], 0.0)
    h2 = jnp.dot(out1, w2_ref[...], preferred_element_type=jnp.float32)
    h2_ref[...] = h2
    g2_ref[...] = h2 * dis


_mid = pl.pallas_call(
    _mid_body,
    grid=(GRID,),
    in_specs=[
        pl.BlockSpec((2, BLK, 16), lambda i: (0, i, 0)),
        pl.BlockSpec((BLK, 32), lambda i: (i, 0)),
        pl.BlockSpec((BLK, 1), lambda i: (i, 0)),
        pl.BlockSpec((BLK, 1), lambda i: (i, 0)),
        pl.BlockSpec((1, 32), lambda i: (0, 0)),
        pl.BlockSpec((32, 16), lambda i: (0, 0)),
    ],
    out_specs=[
        pl.BlockSpec((BLK, 16), lambda i: (i, 0)),
        pl.BlockSpec((BLK, 16), lambda i: (i, 0)),
    ],
    out_shape=[
        jax.ShapeDtypeStruct((N_PAD, 16), jnp.float32),
        jax.ShapeDtypeStruct((N_PAD, 16), jnp.float32),
    ],
)


def _fin_body(acc_ref, h2_ref, dega_ref, degb_ref, b2_ref, out_ref):
    dis = lax.rsqrt(dega_ref[...] + degb_ref[...] + 1.0)
    s = acc_ref[0] + acc_ref[1]
    out_ref[...] = dis * s + (dis * dis) * h2_ref[...] + b2_ref[...]


_fin = pl.pallas_call(
    _fin_body,
    grid=(GRID,),
    in_specs=[
        pl.BlockSpec((2, BLK, 16), lambda i: (0, i, 0)),
        pl.BlockSpec((BLK, 16), lambda i: (i, 0)),
        pl.BlockSpec((BLK, 1), lambda i: (i, 0)),
        pl.BlockSpec((BLK, 1), lambda i: (i, 0)),
        pl.BlockSpec((1, 16), lambda i: (0, 0)),
    ],
    out_specs=pl.BlockSpec((BLK, 16), lambda i: (i, 0)),
    out_shape=jax.ShapeDtypeStruct((N, 16), jnp.float32),
)


def kernel(x, edge_index, W1, b1, W2, b2):
    src = edge_index[0].astype(jnp.int32)
    dst = edge_index[1].astype(jnp.int32)
    pad = jnp.full((E_PAD - E,), SENT, jnp.int32)
    src_rows = jnp.concatenate([src, pad]).reshape(ROWS, LANES)
    dst_rows = jnp.concatenate([dst, pad]).reshape(ROWS, LANES)
    ones = jnp.ones((LANES,), jnp.float32)
    zeros1 = jnp.zeros((N_PAD,), jnp.float32)
    zeros2 = jnp.zeros((N_PAD, 16), jnp.float32)

    degs = _deg_call(dst_rows, ones, zeros1)
    dega = degs[0, :N].reshape(N, 1)
    degb = degs[1, :N].reshape(N, 1)

    h1, g0, g1 = _prep1(x, W1, dega, degb)
    acc1 = _pass_b(g0, g1, src_rows, dst_rows, zeros2)
    h2, g2 = _mid(acc1, h1, dega, degb, b1.reshape(1, 32), W2)
    acc2 = _pass_c(g2, src_rows, dst_rows, zeros2)
    return _fin(acc2, h2, dega, degb, b2.reshape(1, 16))


# speedup vs baseline: 29.1188x; 29.1188x over previous
import functools

import jax
import jax.numpy as jnp
from jax import lax
from jax.experimental import pallas as pl
from jax.experimental.pallas import tpu as pltpu
from jax.experimental.pallas import tpu_sc as plsc

N = 100000
E = 3200000
NC, NS = 2, 16
LANES = 128
E_PAD = 3276800
ROWS = E_PAD // LANES
RPT = 6400
SUB = 800
N_PAD = NS * RPT
SENT = N_PAD - 1
CHUNK = 8
BLK = 2000
GRID = N // BLK

_mesh = plsc.VectorSubcoreMesh(core_axis_name="c", subcore_axis_name="s")
_sc_params = pltpu.CompilerParams(use_tc_tiling_on_sc=False)


def _deg_body(dst_rows, out_hbm, idx_v, ones_v, bounce, deg_sh):
    cid = lax.axis_index("c")
    sid = lax.axis_index("s")

    def initv(i, carry):
        ones_v[pl.ds(i * 16, 16)] = jnp.ones((16,), jnp.float32)
        return carry

    lax.fori_loop(0, LANES // 16, initv, 0)

    def initb(i, carry):
        bounce[pl.ds(i * 16, 16)] = jnp.zeros((16,), jnp.float32)
        return carry

    lax.fori_loop(0, SUB // 16, initb, 0)
    for k in range(RPT // SUB):
        pltpu.sync_copy(bounce, deg_sh.at[pl.ds(sid * RPT + k * SUB, SUB)])
    plsc.subcore_barrier()
    rows_per_tile = ROWS // (NC * NS)
    base = (cid * NS + sid) * rows_per_tile

    def chunk(i, carry):
        r0 = base + i * CHUNK
        pltpu.sync_copy(dst_rows.at[pl.ds(r0, CHUNK)], idx_v)
        for j in range(CHUNK):
            pltpu.sync_copy(ones_v, deg_sh.at[idx_v.at[j]], add=True)
        return carry

    lax.fori_loop(0, rows_per_tile // CHUNK, chunk, 0)
    plsc.subcore_barrier()
    for k in range(RPT // SUB):
        pltpu.sync_copy(deg_sh.at[pl.ds(sid * RPT + k * SUB, SUB)], bounce)
        pltpu.sync_copy(
            bounce,
            out_hbm.at[pl.ds(cid * N_PAD + sid * RPT + k * SUB, SUB)])


_deg_call = pl.kernel(
    _deg_body,
    out_type=jax.ShapeDtypeStruct((NC * N_PAD,), jnp.float32),
    mesh=_mesh,
    compiler_params=_sc_params,
    scratch_types=[
        pltpu.VMEM((CHUNK, LANES), jnp.int32),
        pltpu.VMEM((LANES,), jnp.float32),
        pltpu.VMEM((SUB,), jnp.float32),
        pltpu.VMEM_SHARED((N_PAD,), jnp.float32),
    ],
)


def _edge_pass_body(dual, *refs):
    if dual:
        (t0, t1, src_rows, dst_rows, out_hbm,
         sidx, didx, rows_v, acc_sh, sem) = refs
    else:
        (t0, src_rows, dst_rows, out_hbm,
         sidx, didx, rows_v, acc_sh, sem) = refs
        t1 = t0
    cid = lax.axis_index("c")
    sid = lax.axis_index("s")

    def initz(i, carry):
        rows_v[i] = jnp.zeros((16,), jnp.float32)
        return carry

    lax.fori_loop(0, SUB, initz, 0)
    for k in range(RPT // SUB):
        pltpu.sync_copy(rows_v.at[pl.ds(0, SUB)],
                        acc_sh.at[pl.ds(sid * RPT + k * SUB, SUB)])
    plsc.subcore_barrier()
    if dual:
        rows_per_tile = ROWS // NS
        base = sid * rows_per_tile
    else:
        rows_per_tile = ROWS // (NC * NS)
        base = (cid * NS + sid) * rows_per_tile

    def chunk(i, carry):
        r0 = base + i * CHUNK
        pltpu.sync_copy(src_rows.at[pl.ds(r0, CHUNK)], sidx)
        pltpu.sync_copy(dst_rows.at[pl.ds(r0, CHUNK)], didx)

        def gather(tbl):
            copies = [
                pltpu.async_copy(tbl.at[sidx.at[j]],
                                 rows_v.at[pl.ds(j * LANES, LANES)], sem)
                for j in range(CHUNK)
            ]
            for c in copies:
                c.wait()

        if dual:
            @pl.when(cid == 0)
            def _():
                gather(t0)

            @pl.when(cid == 1)
            def _():
                gather(t1)
        else:
            gather(t0)
        for j in range(CHUNK):
            pltpu.sync_copy(rows_v.at[pl.ds(j * LANES, LANES)],
                            acc_sh.at[didx.at[j]], add=True)
        return carry

    lax.fori_loop(0, rows_per_tile // CHUNK, chunk, 0)
    plsc.subcore_barrier()
    for k in range(RPT // SUB):
        pltpu.sync_copy(acc_sh.at[pl.ds(sid * RPT + k * SUB, SUB)],
                        rows_v.at[pl.ds(0, SUB)])
        pltpu.sync_copy(
            rows_v.at[pl.ds(0, SUB)],
            out_hbm.at[pl.ds(cid * N_PAD + sid * RPT + k * SUB, SUB)])


def _make_edge_pass(dual):
    return pl.kernel(
        functools.partial(_edge_pass_body, dual),
        out_type=jax.ShapeDtypeStruct((NC * N_PAD, 16), jnp.float32),
        mesh=_mesh,
        compiler_params=_sc_params,
        scratch_types=[
            pltpu.VMEM((CHUNK, LANES), jnp.int32),
            pltpu.VMEM((CHUNK, LANES), jnp.int32),
            pltpu.VMEM((CHUNK * LANES, 16), jnp.float32),
            pltpu.VMEM_SHARED((N_PAD, 16), jnp.float32),
            pltpu.SemaphoreType.DMA,
        ],
    )


_pass_b = _make_edge_pass(True)
_pass_c = _make_edge_pass(False)


def _prep1_body(x_ref, w_ref, dega_ref, degb_ref, h_ref, g0_ref, g1_ref):
    dis = lax.rsqrt(dega_ref[...] + degb_ref[...] + 1.0)
    h = jnp.dot(x_ref[...], w_ref[...], preferred_element_type=jnp.float32)
    h_ref[...] = h
    g = h * dis
    g0_ref[...] = g[:, :16]
    g1_ref[...] = g[:, 16:]


_prep1 = pl.pallas_call(
    _prep1_body,
    grid=(GRID,),
    in_specs=[
        pl.BlockSpec((BLK, 16), lambda i: (i, 0)),
        pl.BlockSpec((16, 32), lambda i: (0, 0)),
        pl.BlockSpec((BLK, 1), lambda i: (i, 0)),
        pl.BlockSpec((BLK, 1), lambda i: (i, 0)),
    ],
    out_specs=[
        pl.BlockSpec((BLK, 32), lambda i: (i, 0)),
        pl.BlockSpec((BLK, 16), lambda i: (i, 0)),
        pl.BlockSpec((BLK, 16), lambda i: (i, 0)),
    ],
    out_shape=[
        jax.ShapeDtypeStruct((N_PAD, 32), jnp.float32),
        jax.ShapeDtypeStruct((N_PAD, 16), jnp.float32),
        jax.ShapeDtypeStruct((N_PAD, 16), jnp.float32),
    ],
)


def _mid_body(acc_ref, h1_ref, dega_ref, degb_ref, b1_ref, w2_ref,
              h2_ref, g2_ref):
    dis = lax.rsqrt(dega_ref[...] + degb_ref[...] + 1.0)
    accc = jnp.concatenate([acc_ref[0], acc_ref[1]], axis=1)
    out1 = jnp.maximum(
        dis * accc + (dis * dis) * h1_ref[...] + b1_ref[...], 0.0)
    h2 = jnp.dot(out1, w2_ref[...], preferred_element_type=jnp.float32)
    h2_ref[...] = h2
    g2_ref[...] = h2 * dis


_mid = pl.pallas_call(
    _mid_body,
    grid=(GRID,),
    in_specs=[
        pl.BlockSpec((2, BLK, 16), lambda i: (0, i, 0)),
        pl.BlockSpec((BLK, 32), lambda i: (i, 0)),
        pl.BlockSpec((BLK, 1), lambda i: (i, 0)),
        pl.BlockSpec((BLK, 1), lambda i: (i, 0)),
        pl.BlockSpec((1, 32), lambda i: (0, 0)),
        pl.BlockSpec((32, 16), lambda i: (0, 0)),
    ],
    out_specs=[
        pl.BlockSpec((BLK, 16), lambda i: (i, 0)),
        pl.BlockSpec((BLK, 16), lambda i: (i, 0)),
    ],
    out_shape=[
        jax.ShapeDtypeStruct((N_PAD, 16), jnp.float32),
        jax.ShapeDtypeStruct((N_PAD, 16), jnp.float32),
    ],
)


def _fin_body(acc_ref, h2_ref, dega_ref, degb_ref, b2_ref, out_ref):
    dis = lax.rsqrt(dega_ref[...] + degb_ref[...] + 1.0)
    s = acc_ref[0] + acc_ref[1]
    out_ref[...] = dis * s + (dis * dis) * h2_ref[...] + b2_ref[...]


_fin = pl.pallas_call(
    _fin_body,
    grid=(GRID,),
    in_specs=[
        pl.BlockSpec((2, BLK, 16), lambda i: (0, i, 0)),
        pl.BlockSpec((BLK, 16), lambda i: (i, 0)),
        pl.BlockSpec((BLK, 1), lambda i: (i, 0)),
        pl.BlockSpec((BLK, 1), lambda i: (i, 0)),
        pl.BlockSpec((1, 16), lambda i: (0, 0)),
    ],
    out_specs=pl.BlockSpec((BLK, 16), lambda i: (i, 0)),
    out_shape=jax.ShapeDtypeStruct((N, 16), jnp.float32),
)


def kernel(x, edge_index, W1, b1, W2, b2):
    src = edge_index[0].astype(jnp.int32)
    dst = edge_index[1].astype(jnp.int32)
    pad_src = jnp.full((E_PAD - E,), SENT, jnp.int32)
    pad_dst = N + jnp.arange(E_PAD - E, dtype=jnp.int32) % (N_PAD - N)
    src_rows = jnp.concatenate([src, pad_src]).reshape(ROWS, LANES)
    dst_rows = jnp.concatenate([dst, pad_dst]).reshape(ROWS, LANES)
    degs = _deg_call(dst_rows).reshape(NC, N_PAD)
    dega = degs[0, :N].reshape(N, 1)
    degb = degs[1, :N].reshape(N, 1)

    h1, g0, g1 = _prep1(x, W1, dega, degb)
    acc1 = _pass_b(g0, g1, src_rows, dst_rows)
    acc1 = acc1.reshape(NC, N_PAD, 16)
    h2, g2 = _mid(acc1, h1, dega, degb, b1.reshape(1, 32), W2)
    acc2 = _pass_c(g2, src_rows, dst_rows)
    acc2 = acc2.reshape(NC, N_PAD, 16)
    return _fin(acc2, h2, dega, degb, b2.reshape(1, 16))


# speedup vs baseline: 34.3418x; 1.1794x over previous
import functools

import jax
import jax.numpy as jnp
from jax import lax
from jax.experimental import pallas as pl
from jax.experimental.pallas import tpu as pltpu
from jax.experimental.pallas import tpu_sc as plsc

N = 100000
E = 3200000
NC, NS = 2, 16
LANES = 128
E_PAD = 3276800
ROWS = E_PAD // LANES
RPT = 6400
SUB = 800
N_PAD = NS * RPT
SENT = N_PAD - 1
CHUNK = 4
SUPER = 40
WSUB = 400
BLK = 2000
GRID = N // BLK

_mesh = plsc.VectorSubcoreMesh(core_axis_name="c", subcore_axis_name="s")
_sc_params = pltpu.CompilerParams(use_tc_tiling_on_sc=False)


def _deg_body(dst_rows, out_hbm, idx_v, ones_v, bounce, deg_sh):
    cid = lax.axis_index("c")
    sid = lax.axis_index("s")

    def initv(i, carry):
        ones_v[pl.ds(i * 16, 16)] = jnp.ones((16,), jnp.float32)
        return carry

    lax.fori_loop(0, LANES // 16, initv, 0)

    def initb(i, carry):
        bounce[pl.ds(i * 16, 16)] = jnp.zeros((16,), jnp.float32)
        return carry

    lax.fori_loop(0, SUB // 16, initb, 0)
    for k in range(RPT // SUB):
        pltpu.sync_copy(bounce, deg_sh.at[pl.ds(sid * RPT + k * SUB, SUB)])
    plsc.subcore_barrier()
    rows_per_tile = ROWS // (NC * NS)
    base = (cid * NS + sid) * rows_per_tile

    def chunk(i, carry):
        r0 = base + i * CHUNK
        pltpu.sync_copy(dst_rows.at[pl.ds(r0, CHUNK)], idx_v)
        for j in range(CHUNK):
            pltpu.sync_copy(ones_v, deg_sh.at[idx_v.at[j]], add=True)
        return carry

    lax.fori_loop(0, rows_per_tile // CHUNK, chunk, 0)
    plsc.subcore_barrier()
    for k in range(RPT // SUB):
        pltpu.sync_copy(deg_sh.at[pl.ds(sid * RPT + k * SUB, SUB)], bounce)
        pltpu.sync_copy(
            bounce,
            out_hbm.at[pl.ds(cid * N_PAD + sid * RPT + k * SUB, SUB)])


_deg_call = pl.kernel(
    _deg_body,
    out_type=jax.ShapeDtypeStruct((NC * N_PAD,), jnp.float32),
    mesh=_mesh,
    compiler_params=_sc_params,
    scratch_types=[
        pltpu.VMEM((CHUNK, LANES), jnp.int32),
        pltpu.VMEM((LANES,), jnp.float32),
        pltpu.VMEM((SUB,), jnp.float32),
        pltpu.VMEM_SHARED((N_PAD,), jnp.float32),
    ],
)


def _edge_pass_body(dual, *refs):
    if dual:
        (t0, t1, src_rows, dst_rows, out_hbm,
         sidx, didx, rows_a, rows_b, acc_sh, sem_a, sem_b) = refs
    else:
        (t0, src_rows, dst_rows, out_hbm,
         sidx, didx, rows_a, rows_b, acc_sh, sem_a, sem_b) = refs
        t1 = t0
    cid = lax.axis_index("c")
    sid = lax.axis_index("s")

    def initz(i, carry):
        rows_a[i] = jnp.zeros((16,), jnp.float32)
        return carry

    lax.fori_loop(0, WSUB, initz, 0)
    for k in range(RPT // WSUB):
        pltpu.sync_copy(rows_a.at[pl.ds(0, WSUB)],
                        acc_sh.at[pl.ds(sid * RPT + k * WSUB, WSUB)])
    plsc.subcore_barrier()
    if dual:
        rows_per_tile = ROWS // NS
        base = sid * rows_per_tile
    else:
        rows_per_tile = ROWS // (NC * NS)
        base = (cid * NS + sid) * rows_per_tile

    nchunk = SUPER // CHUNK
    bufs = [(rows_a, sem_a), (rows_b, sem_b)]

    def superchunk(i, carry):
        s0 = base + i * SUPER
        pltpu.sync_copy(src_rows.at[pl.ds(s0, SUPER)], sidx)
        pltpu.sync_copy(dst_rows.at[pl.ds(s0, SUPER)], didx)

        def issue(c, tbl):
            buf, sem = bufs[c % 2]
            return [
                pltpu.async_copy(tbl.at[sidx.at[c * CHUNK + j]],
                                 buf.at[pl.ds(j * LANES, LANES)], sem)
                for j in range(CHUNK)
            ]

        def run(tbl):
            pending = issue(0, tbl)
            for c in range(nchunk):
                nxt = issue(c + 1, tbl) if c + 1 < nchunk else []
                for h in pending:
                    h.wait()
                buf, _ = bufs[c % 2]
                for j in range(CHUNK):
                    pltpu.sync_copy(buf.at[pl.ds(j * LANES, LANES)],
                                    acc_sh.at[didx.at[c * CHUNK + j]],
                                    add=True)
                pending = nxt

        if dual:
            @pl.when(cid == 0)
            def _():
                run(t0)

            @pl.when(cid == 1)
            def _():
                run(t1)
        else:
            run(t0)
        return carry

    lax.fori_loop(0, rows_per_tile // SUPER, superchunk, 0)
    plsc.subcore_barrier()
    for k in range(RPT // WSUB):
        pltpu.sync_copy(acc_sh.at[pl.ds(sid * RPT + k * WSUB, WSUB)],
                        rows_a.at[pl.ds(0, WSUB)])
        pltpu.sync_copy(
            rows_a.at[pl.ds(0, WSUB)],
            out_hbm.at[pl.ds(cid * N_PAD + sid * RPT + k * WSUB, WSUB)])


def _make_edge_pass(dual):
    return pl.kernel(
        functools.partial(_edge_pass_body, dual),
        out_type=jax.ShapeDtypeStruct((NC * N_PAD, 16), jnp.float32),
        mesh=_mesh,
        compiler_params=_sc_params,
        scratch_types=[
            pltpu.VMEM((SUPER, LANES), jnp.int32),
            pltpu.VMEM((SUPER, LANES), jnp.int32),
            pltpu.VMEM((CHUNK * LANES, 16), jnp.float32),
            pltpu.VMEM((CHUNK * LANES, 16), jnp.float32),
            pltpu.VMEM_SHARED((N_PAD, 16), jnp.float32),
            pltpu.SemaphoreType.DMA,
            pltpu.SemaphoreType.DMA,
        ],
    )


_pass_b = _make_edge_pass(True)
_pass_c = _make_edge_pass(False)


def _prep1_body(x_ref, w_ref, dega_ref, degb_ref, h_ref, g0_ref, g1_ref):
    dis = lax.rsqrt(dega_ref[...] + degb_ref[...] + 1.0)
    h = jnp.dot(x_ref[...], w_ref[...], preferred_element_type=jnp.float32)
    h_ref[...] = h
    g = h * dis
    g0_ref[...] = g[:, :16]
    g1_ref[...] = g[:, 16:]


_prep1 = pl.pallas_call(
    _prep1_body,
    grid=(GRID,),
    in_specs=[
        pl.BlockSpec((BLK, 16), lambda i: (i, 0)),
        pl.BlockSpec((16, 32), lambda i: (0, 0)),
        pl.BlockSpec((BLK, 1), lambda i: (i, 0)),
        pl.BlockSpec((BLK, 1), lambda i: (i, 0)),
    ],
    out_specs=[
        pl.BlockSpec((BLK, 32), lambda i: (i, 0)),
        pl.BlockSpec((BLK, 16), lambda i: (i, 0)),
        pl.BlockSpec((BLK, 16), lambda i: (i, 0)),
    ],
    out_shape=[
        jax.ShapeDtypeStruct((N_PAD, 32), jnp.float32),
        jax.ShapeDtypeStruct((N_PAD, 16), jnp.float32),
        jax.ShapeDtypeStruct((N_PAD, 16), jnp.float32),
    ],
)


def _mid_body(acc_ref, h1_ref, dega_ref, degb_ref, b1_ref, w2_ref,
              h2_ref, g2_ref):
    dis = lax.rsqrt(dega_ref[...] + degb_ref[...] + 1.0)
    accc = jnp.concatenate([acc_ref[0], acc_ref[1]], axis=1)
    out1 = jnp.maximum(
        dis * accc + (dis * dis) * h1_ref[...] + b1_ref[...], 0.0)
    h2 = jnp.dot(out1, w2_ref[...], preferred_element_type=jnp.float32)
    h2_ref[...] = h2
    g2_ref[...] = h2 * dis


_mid = pl.pallas_call(
    _mid_body,
    grid=(GRID,),
    in_specs=[
        pl.BlockSpec((2, BLK, 16), lambda i: (0, i, 0)),
        pl.BlockSpec((BLK, 32), lambda i: (i, 0)),
        pl.BlockSpec((BLK, 1), lambda i: (i, 0)),
        pl.BlockSpec((BLK, 1), lambda i: (i, 0)),
        pl.BlockSpec((1, 32), lambda i: (0, 0)),
        pl.BlockSpec((32, 16), lambda i: (0, 0)),
    ],
    out_specs=[
        pl.BlockSpec((BLK, 16), lambda i: (i, 0)),
        pl.BlockSpec((BLK, 16), lambda i: (i, 0)),
    ],
    out_shape=[
        jax.ShapeDtypeStruct((N_PAD, 16), jnp.float32),
        jax.ShapeDtypeStruct((N_PAD, 16), jnp.float32),
    ],
)


def _fin_body(acc_ref, h2_ref, dega_ref, degb_ref, b2_ref, out_ref):
    dis = lax.rsqrt(dega_ref[...] + degb_ref[...] + 1.0)
    s = acc_ref[0] + acc_ref[1]
    out_ref[...] = dis * s + (dis * dis) * h2_ref[...] + b2_ref[...]


_fin = pl.pallas_call(
    _fin_body,
    grid=(GRID,),
    in_specs=[
        pl.BlockSpec((2, BLK, 16), lambda i: (0, i, 0)),
        pl.BlockSpec((BLK, 16), lambda i: (i, 0)),
        pl.BlockSpec((BLK, 1), lambda i: (i, 0)),
        pl.BlockSpec((BLK, 1), lambda i: (i, 0)),
        pl.BlockSpec((1, 16), lambda i: (0, 0)),
    ],
    out_specs=pl.BlockSpec((BLK, 16), lambda i: (i, 0)),
    out_shape=jax.ShapeDtypeStruct((N, 16), jnp.float32),
)


def kernel(x, edge_index, W1, b1, W2, b2):
    src = edge_index[0].astype(jnp.int32)
    dst = edge_index[1].astype(jnp.int32)
    pad_src = jnp.full((E_PAD - E,), SENT, jnp.int32)
    pad_dst = N + jnp.arange(E_PAD - E, dtype=jnp.int32) % (N_PAD - N)
    src_rows = jnp.concatenate([src, pad_src]).reshape(ROWS, LANES)
    dst_rows = jnp.concatenate([dst, pad_dst]).reshape(ROWS, LANES)
    degs = _deg_call(dst_rows).reshape(NC, N_PAD)
    dega = degs[0, :N].reshape(N, 1)
    degb = degs[1, :N].reshape(N, 1)

    h1, g0, g1 = _prep1(x, W1, dega, degb)
    acc1 = _pass_b(g0, g1, src_rows, dst_rows)
    acc1 = acc1.reshape(NC, N_PAD, 16)
    h2, g2 = _mid(acc1, h1, dega, degb, b1.reshape(1, 32), W2)
    acc2 = _pass_c(g2, src_rows, dst_rows)
    acc2 = acc2.reshape(NC, N_PAD, 16)
    return _fin(acc2, h2, dega, degb, b2.reshape(1, 16))


# speedup vs baseline: 57.8706x; 1.6851x over previous
import functools

import jax
import jax.numpy as jnp
from jax import lax
from jax.experimental import pallas as pl
from jax.experimental.pallas import tpu as pltpu
from jax.experimental.pallas import tpu_sc as plsc

N = 100000
E = 3200000
NC, NS = 2, 16
LANES = 128
E_PAD = 3276800
ROWS = E_PAD // LANES
RPT = 6400
SUB = 800
N_PAD = NS * RPT
SENT = N_PAD - 1
CHUNK = 4
SUPER = 40
WSUB = 400
BLK = 2000
GRID = N // BLK

_mesh = plsc.VectorSubcoreMesh(core_axis_name="c", subcore_axis_name="s")
_sc_params = pltpu.CompilerParams(use_tc_tiling_on_sc=False)


def _deg_body(dst_rows, out_hbm, idx_v, ones_v, bounce, deg_sh):
    cid = lax.axis_index("c")
    sid = lax.axis_index("s")

    def initv(i, carry):
        ones_v[pl.ds(i * 16, 16)] = jnp.ones((16,), jnp.float32)
        return carry

    lax.fori_loop(0, LANES // 16, initv, 0)

    def initb(i, carry):
        bounce[pl.ds(i * 16, 16)] = jnp.zeros((16,), jnp.float32)
        return carry

    lax.fori_loop(0, SUB // 16, initb, 0)
    for k in range(RPT // SUB):
        pltpu.sync_copy(bounce, deg_sh.at[pl.ds(sid * RPT + k * SUB, SUB)])
    plsc.subcore_barrier()
    rows_per_tile = ROWS // (NC * NS)
    base = (cid * NS + sid) * rows_per_tile

    def chunk(i, carry):
        r0 = base + i * CHUNK
        pltpu.sync_copy(dst_rows.at[pl.ds(r0, CHUNK)], idx_v)
        for j in range(CHUNK):
            pltpu.sync_copy(ones_v, deg_sh.at[idx_v.at[j]], add=True)
        return carry

    lax.fori_loop(0, rows_per_tile // CHUNK, chunk, 0)
    plsc.subcore_barrier()
    for k in range(RPT // SUB):
        pltpu.sync_copy(deg_sh.at[pl.ds(sid * RPT + k * SUB, SUB)], bounce)
        pltpu.sync_copy(
            bounce,
            out_hbm.at[pl.ds(cid * N_PAD + sid * RPT + k * SUB, SUB)])


_deg_call = pl.kernel(
    _deg_body,
    out_type=jax.ShapeDtypeStruct((NC * N_PAD,), jnp.float32),
    mesh=_mesh,
    compiler_params=_sc_params,
    scratch_types=[
        pltpu.VMEM((CHUNK, LANES), jnp.int32),
        pltpu.VMEM((LANES,), jnp.float32),
        pltpu.VMEM((SUB,), jnp.float32),
        pltpu.VMEM_SHARED((N_PAD,), jnp.float32),
    ],
)


def _edge_pass_body(dual, *refs):
    if dual:
        (t0, t1, src_rows, dst_rows, out_hbm, sidx, didx, rows_a, rows_b,
         acc_sh, sem_a, sem_b, sem_sa, sem_sb) = refs
    else:
        (t0, src_rows, dst_rows, out_hbm, sidx, didx, rows_a, rows_b,
         acc_sh, sem_a, sem_b, sem_sa, sem_sb) = refs
        t1 = t0
    ssems = [sem_sa, sem_sb]
    cid = lax.axis_index("c")
    sid = lax.axis_index("s")

    def initz(i, carry):
        rows_a[i] = jnp.zeros((16,), jnp.float32)
        return carry

    lax.fori_loop(0, WSUB, initz, 0)
    for k in range(RPT // WSUB):
        pltpu.sync_copy(rows_a.at[pl.ds(0, WSUB)],
                        acc_sh.at[pl.ds(sid * RPT + k * WSUB, WSUB)])
    plsc.subcore_barrier()
    if dual:
        rows_per_tile = ROWS // NS
        base = sid * rows_per_tile
    else:
        rows_per_tile = ROWS // (NC * NS)
        base = (cid * NS + sid) * rows_per_tile

    nchunk = SUPER // CHUNK
    bufs = [(rows_a, sem_a), (rows_b, sem_b)]

    def superchunk(i, carry):
        s0 = base + i * SUPER
        pltpu.sync_copy(src_rows.at[pl.ds(s0, SUPER)], sidx)
        pltpu.sync_copy(dst_rows.at[pl.ds(s0, SUPER)], didx)

        def issue(c, tbl):
            buf, sem = bufs[c % 2]
            return [
                pltpu.async_copy(tbl.at[sidx.at[c * CHUNK + j]],
                                 buf.at[pl.ds(j * LANES, LANES)], sem)
                for j in range(CHUNK)
            ]

        def run(tbl):
            pending = issue(0, tbl)
            scat = {}
            for c in range(nchunk):
                if c + 1 < nchunk:
                    for h in scat.pop(c - 1, ()):
                        h.wait()
                    nxt = issue(c + 1, tbl)
                else:
                    nxt = []
                for h in pending:
                    h.wait()
                buf, _ = bufs[c % 2]
                ssem = ssems[c % 2]
                scat[c] = [
                    pltpu.async_copy(buf.at[pl.ds(j * LANES, LANES)],
                                     acc_sh.at[didx.at[c * CHUNK + j]],
                                     ssem, add=True)
                    for j in range(CHUNK)
                ]
                pending = nxt
            for hs in scat.values():
                for h in hs:
                    h.wait()

        if dual:
            @pl.when(cid == 0)
            def _():
                run(t0)

            @pl.when(cid == 1)
            def _():
                run(t1)
        else:
            run(t0)
        return carry

    lax.fori_loop(0, rows_per_tile // SUPER, superchunk, 0)
    plsc.subcore_barrier()
    for k in range(RPT // WSUB):
        pltpu.sync_copy(acc_sh.at[pl.ds(sid * RPT + k * WSUB, WSUB)],
                        rows_a.at[pl.ds(0, WSUB)])
        pltpu.sync_copy(
            rows_a.at[pl.ds(0, WSUB)],
            out_hbm.at[pl.ds(cid * N_PAD + sid * RPT + k * WSUB, WSUB)])


def _make_edge_pass(dual):
    return pl.kernel(
        functools.partial(_edge_pass_body, dual),
        out_type=jax.ShapeDtypeStruct((NC * N_PAD, 16), jnp.float32),
        mesh=_mesh,
        compiler_params=_sc_params,
        scratch_types=[
            pltpu.VMEM((SUPER, LANES), jnp.int32),
            pltpu.VMEM((SUPER, LANES), jnp.int32),
            pltpu.VMEM((CHUNK * LANES, 16), jnp.float32),
            pltpu.VMEM((CHUNK * LANES, 16), jnp.float32),
            pltpu.VMEM_SHARED((N_PAD, 16), jnp.float32),
            pltpu.SemaphoreType.DMA,
            pltpu.SemaphoreType.DMA,
            pltpu.SemaphoreType.DMA,
            pltpu.SemaphoreType.DMA,
        ],
    )


_pass_b = _make_edge_pass(True)
_pass_c = _make_edge_pass(False)


def _prep1_body(x_ref, w_ref, dega_ref, degb_ref, h_ref, g0_ref, g1_ref):
    dis = lax.rsqrt(dega_ref[...] + degb_ref[...] + 1.0)
    h = jnp.dot(x_ref[...], w_ref[...], preferred_element_type=jnp.float32)
    h_ref[...] = h
    g = h * dis
    g0_ref[...] = g[:, :16]
    g1_ref[...] = g[:, 16:]


_prep1 = pl.pallas_call(
    _prep1_body,
    grid=(GRID,),
    in_specs=[
        pl.BlockSpec((BLK, 16), lambda i: (i, 0)),
        pl.BlockSpec((16, 32), lambda i: (0, 0)),
        pl.BlockSpec((BLK, 1), lambda i: (i, 0)),
        pl.BlockSpec((BLK, 1), lambda i: (i, 0)),
    ],
    out_specs=[
        pl.BlockSpec((BLK, 32), lambda i: (i, 0)),
        pl.BlockSpec((BLK, 16), lambda i: (i, 0)),
        pl.BlockSpec((BLK, 16), lambda i: (i, 0)),
    ],
    out_shape=[
        jax.ShapeDtypeStruct((N_PAD, 32), jnp.float32),
        jax.ShapeDtypeStruct((N_PAD, 16), jnp.float32),
        jax.ShapeDtypeStruct((N_PAD, 16), jnp.float32),
    ],
)


def _mid_body(acc_ref, h1_ref, dega_ref, degb_ref, b1_ref, w2_ref,
              h2_ref, g2_ref):
    dis = lax.rsqrt(dega_ref[...] + degb_ref[...] + 1.0)
    accc = jnp.concatenate([acc_ref[0], acc_ref[1]], axis=1)
    out1 = jnp.maximum(
        dis * accc + (dis * dis) * h1_ref[...] + b1_ref[...], 0.0)
    h2 = jnp.dot(out1, w2_ref[...], preferred_element_type=jnp.float32)
    h2_ref[...] = h2
    g2_ref[...] = h2 * dis


_mid = pl.pallas_call(
    _mid_body,
    grid=(GRID,),
    in_specs=[
        pl.BlockSpec((2, BLK, 16), lambda i: (0, i, 0)),
        pl.BlockSpec((BLK, 32), lambda i: (i, 0)),
        pl.BlockSpec((BLK, 1), lambda i: (i, 0)),
        pl.BlockSpec((BLK, 1), lambda i: (i, 0)),
        pl.BlockSpec((1, 32), lambda i: (0, 0)),
        pl.BlockSpec((32, 16), lambda i: (0, 0)),
    ],
    out_specs=[
        pl.BlockSpec((BLK, 16), lambda i: (i, 0)),
        pl.BlockSpec((BLK, 16), lambda i: (i, 0)),
    ],
    out_shape=[
        jax.ShapeDtypeStruct((N_PAD, 16), jnp.float32),
        jax.ShapeDtypeStruct((N_PAD, 16), jnp.float32),
    ],
)


def _fin_body(acc_ref, h2_ref, dega_ref, degb_ref, b2_ref, out_ref):
    dis = lax.rsqrt(dega_ref[...] + degb_ref[...] + 1.0)
    s = acc_ref[0] + acc_ref[1]
    out_ref[...] = dis * s + (dis * dis) * h2_ref[...] + b2_ref[...]


_fin = pl.pallas_call(
    _fin_body,
    grid=(GRID,),
    in_specs=[
        pl.BlockSpec((2, BLK, 16), lambda i: (0, i, 0)),
        pl.BlockSpec((BLK, 16), lambda i: (i, 0)),
        pl.BlockSpec((BLK, 1), lambda i: (i, 0)),
        pl.BlockSpec((BLK, 1), lambda i: (i, 0)),
        pl.BlockSpec((1, 16), lambda i: (0, 0)),
    ],
    out_specs=pl.BlockSpec((BLK, 16), lambda i: (i, 0)),
    out_shape=jax.ShapeDtypeStruct((N, 16), jnp.float32),
)


def kernel(x, edge_index, W1, b1, W2, b2):
    src = edge_index[0].astype(jnp.int32)
    dst = edge_index[1].astype(jnp.int32)
    pad_src = jnp.arange(E_PAD - E, dtype=jnp.int32) % 2048
    pad_dst = N + jnp.arange(E_PAD - E, dtype=jnp.int32) % (N_PAD - N)
    src_rows = jnp.concatenate([src, pad_src]).reshape(ROWS, LANES)
    dst_rows = jnp.concatenate([dst, pad_dst]).reshape(ROWS, LANES)
    degs = _deg_call(dst_rows).reshape(NC, N_PAD)
    dega = degs[0, :N].reshape(N, 1)
    degb = degs[1, :N].reshape(N, 1)

    h1, g0, g1 = _prep1(x, W1, dega, degb)
    acc1 = _pass_b(g0, g1, src_rows, dst_rows)
    acc1 = acc1.reshape(NC, N_PAD, 16)
    h2, g2 = _mid(acc1, h1, dega, degb, b1.reshape(1, 32), W2)
    acc2 = _pass_c(g2, src_rows, dst_rows)
    acc2 = acc2.reshape(NC, N_PAD, 16)
    return _fin(acc2, h2, dega, degb, b2.reshape(1, 16))


# speedup vs baseline: 60.8493x; 1.0515x over previous
import functools

import jax
import jax.numpy as jnp
from jax import lax
from jax.experimental import pallas as pl
from jax.experimental.pallas import tpu as pltpu
from jax.experimental.pallas import tpu_sc as plsc

N = 100000
E = 3200000
NC, NS = 2, 16
LANES = 128
E_PAD = 3276800
ROWS = E_PAD // LANES
RPT = 6400
SUB = 800
N_PAD = NS * RPT
SENT = N_PAD - 1
CHUNK = 4
SUPER = 40
WSUB = 400
BLK = 4000
GRID = N // BLK

_mesh = plsc.VectorSubcoreMesh(core_axis_name="c", subcore_axis_name="s")
_sc_params = pltpu.CompilerParams(use_tc_tiling_on_sc=False)


def _deg_body(dst_rows, out_hbm, idx_v, ones_v, bounce, deg_sh):
    cid = lax.axis_index("c")
    sid = lax.axis_index("s")

    def initv(i, carry):
        ones_v[pl.ds(i * 16, 16)] = jnp.ones((16,), jnp.float32)
        return carry

    lax.fori_loop(0, LANES // 16, initv, 0)

    def initb(i, carry):
        bounce[pl.ds(i * 16, 16)] = jnp.zeros((16,), jnp.float32)
        return carry

    lax.fori_loop(0, SUB // 16, initb, 0)
    for k in range(RPT // SUB):
        pltpu.sync_copy(bounce, deg_sh.at[pl.ds(sid * RPT + k * SUB, SUB)])
    plsc.subcore_barrier()
    rows_per_tile = ROWS // (NC * NS)
    base = (cid * NS + sid) * rows_per_tile

    def chunk(i, carry):
        r0 = base + i * CHUNK
        pltpu.sync_copy(dst_rows.at[pl.ds(r0, CHUNK)], idx_v)
        for j in range(CHUNK):
            pltpu.sync_copy(ones_v, deg_sh.at[idx_v.at[j]], add=True)
        return carry

    lax.fori_loop(0, rows_per_tile // CHUNK, chunk, 0)
    plsc.subcore_barrier()
    for k in range(RPT // SUB):
        pltpu.sync_copy(deg_sh.at[pl.ds(sid * RPT + k * SUB, SUB)], bounce)
        pltpu.sync_copy(
            bounce,
            out_hbm.at[pl.ds(cid * N_PAD + sid * RPT + k * SUB, SUB)])


_deg_call = pl.kernel(
    _deg_body,
    out_type=jax.ShapeDtypeStruct((NC * N_PAD,), jnp.float32),
    mesh=_mesh,
    compiler_params=_sc_params,
    scratch_types=[
        pltpu.VMEM((CHUNK, LANES), jnp.int32),
        pltpu.VMEM((LANES,), jnp.float32),
        pltpu.VMEM((SUB,), jnp.float32),
        pltpu.VMEM_SHARED((N_PAD,), jnp.float32),
    ],
)


def _edge_pass_body(dual, *refs):
    if dual:
        (t0, t1, src_rows, dst_rows, out_hbm, sidx, didx, rows_a, rows_b,
         acc_sh, sem_a, sem_b, sem_sa, sem_sb) = refs
    else:
        (t0, src_rows, dst_rows, out_hbm, sidx, didx, rows_a, rows_b,
         acc_sh, sem_a, sem_b, sem_sa, sem_sb) = refs
        t1 = t0
    ssems = [sem_sa, sem_sb]
    cid = lax.axis_index("c")
    sid = lax.axis_index("s")

    def initz(i, carry):
        rows_a[i] = jnp.zeros((16,), jnp.float32)
        return carry

    lax.fori_loop(0, WSUB, initz, 0)
    for k in range(RPT // WSUB):
        pltpu.sync_copy(rows_a.at[pl.ds(0, WSUB)],
                        acc_sh.at[pl.ds(sid * RPT + k * WSUB, WSUB)])
    plsc.subcore_barrier()
    if dual:
        rows_per_tile = ROWS // NS
        base = sid * rows_per_tile
    else:
        rows_per_tile = ROWS // (NC * NS)
        base = (cid * NS + sid) * rows_per_tile

    nchunk = SUPER // CHUNK
    bufs = [(rows_a, sem_a), (rows_b, sem_b)]

    def superchunk(i, carry):
        s0 = base + i * SUPER
        pltpu.sync_copy(src_rows.at[pl.ds(s0, SUPER)], sidx)
        pltpu.sync_copy(dst_rows.at[pl.ds(s0, SUPER)], didx)

        def issue(c, tbl):
            buf, sem = bufs[c % 2]
            return [
                pltpu.async_copy(tbl.at[sidx.at[c * CHUNK + j]],
                                 buf.at[pl.ds(j * LANES, LANES)], sem)
                for j in range(CHUNK)
            ]

        def run(tbl):
            pending = issue(0, tbl)
            scat = {}
            for c in range(nchunk):
                if c + 1 < nchunk:
                    for h in scat.pop(c - 1, ()):
                        h.wait()
                    nxt = issue(c + 1, tbl)
                else:
                    nxt = []
                for h in pending:
                    h.wait()
                buf, _ = bufs[c % 2]
                ssem = ssems[c % 2]
                scat[c] = [
                    pltpu.async_copy(buf.at[pl.ds(j * LANES, LANES)],
                                     acc_sh.at[didx.at[c * CHUNK + j]],
                                     ssem, add=True)
                    for j in range(CHUNK)
                ]
                pending = nxt
            for hs in scat.values():
                for h in hs:
                    h.wait()

        if dual:
            @pl.when(cid == 0)
            def _():
                run(t0)

            @pl.when(cid == 1)
            def _():
                run(t1)
        else:
            run(t0)
        return carry

    lax.fori_loop(0, rows_per_tile // SUPER, superchunk, 0)
    plsc.subcore_barrier()
    for k in range(RPT // WSUB):
        pltpu.sync_copy(acc_sh.at[pl.ds(sid * RPT + k * WSUB, WSUB)],
                        rows_a.at[pl.ds(0, WSUB)])
        pltpu.sync_copy(
            rows_a.at[pl.ds(0, WSUB)],
            out_hbm.at[pl.ds(cid * N_PAD + sid * RPT + k * WSUB, WSUB)])


def _make_edge_pass(dual):
    return pl.kernel(
        functools.partial(_edge_pass_body, dual),
        out_type=jax.ShapeDtypeStruct((NC * N_PAD, 16), jnp.float32),
        mesh=_mesh,
        compiler_params=_sc_params,
        scratch_types=[
            pltpu.VMEM((SUPER, LANES), jnp.int32),
            pltpu.VMEM((SUPER, LANES), jnp.int32),
            pltpu.VMEM((CHUNK * LANES, 16), jnp.float32),
            pltpu.VMEM((CHUNK * LANES, 16), jnp.float32),
            pltpu.VMEM_SHARED((N_PAD, 16), jnp.float32),
            pltpu.SemaphoreType.DMA,
            pltpu.SemaphoreType.DMA,
            pltpu.SemaphoreType.DMA,
            pltpu.SemaphoreType.DMA,
        ],
    )


_pass_b = _make_edge_pass(True)
_pass_c = _make_edge_pass(False)


def _prep1_body(x_ref, w_ref, deg_ref, h_ref, g0_ref, g1_ref):
    dis = lax.rsqrt(deg_ref[...] + 1.0)
    h = jnp.dot(x_ref[...], w_ref[...], preferred_element_type=jnp.float32)
    h_ref[...] = h
    g = h * dis
    g0_ref[...] = g[:, :16]
    g1_ref[...] = g[:, 16:]


_prep1 = pl.pallas_call(
    _prep1_body,
    grid=(GRID,),
    in_specs=[
        pl.BlockSpec((BLK, 16), lambda i: (i, 0)),
        pl.BlockSpec((16, 32), lambda i: (0, 0)),
        pl.BlockSpec((BLK, 1), lambda i: (i, 0)),
    ],
    out_specs=[
        pl.BlockSpec((BLK, 32), lambda i: (i, 0)),
        pl.BlockSpec((BLK, 16), lambda i: (i, 0)),
        pl.BlockSpec((BLK, 16), lambda i: (i, 0)),
    ],
    out_shape=[
        jax.ShapeDtypeStruct((N_PAD, 32), jnp.float32),
        jax.ShapeDtypeStruct((N_PAD, 16), jnp.float32),
        jax.ShapeDtypeStruct((N_PAD, 16), jnp.float32),
    ],
)


def _mid_body(acc_ref, h1_ref, deg_ref, b1_ref, w2_ref, h2_ref, g2_ref):
    dis = lax.rsqrt(deg_ref[...] + 1.0)
    accc = jnp.concatenate([acc_ref[0], acc_ref[1]], axis=1)
    out1 = jnp.maximum(
        dis * accc + (dis * dis) * h1_ref[...] + b1_ref[...], 0.0)
    h2 = jnp.dot(out1, w2_ref[...], preferred_element_type=jnp.float32)
    h2_ref[...] = h2
    g2_ref[...] = h2 * dis


_mid = pl.pallas_call(
    _mid_body,
    grid=(GRID,),
    in_specs=[
        pl.BlockSpec((2, BLK, 16), lambda i: (0, i, 0)),
        pl.BlockSpec((BLK, 32), lambda i: (i, 0)),
        pl.BlockSpec((BLK, 1), lambda i: (i, 0)),
        pl.BlockSpec((1, 32), lambda i: (0, 0)),
        pl.BlockSpec((32, 16), lambda i: (0, 0)),
    ],
    out_specs=[
        pl.BlockSpec((BLK, 16), lambda i: (i, 0)),
        pl.BlockSpec((BLK, 16), lambda i: (i, 0)),
    ],
    out_shape=[
        jax.ShapeDtypeStruct((N_PAD, 16), jnp.float32),
        jax.ShapeDtypeStruct((N_PAD, 16), jnp.float32),
    ],
)


def _fin_body(acc_ref, h2_ref, deg_ref, b2_ref, out_ref):
    dis = lax.rsqrt(deg_ref[...] + 1.0)
    s = acc_ref[0] + acc_ref[1]
    out_ref[...] = dis * s + (dis * dis) * h2_ref[...] + b2_ref[...]


_fin = pl.pallas_call(
    _fin_body,
    grid=(GRID,),
    in_specs=[
        pl.BlockSpec((2, BLK, 16), lambda i: (0, i, 0)),
        pl.BlockSpec((BLK, 16), lambda i: (i, 0)),
        pl.BlockSpec((BLK, 1), lambda i: (i, 0)),
        pl.BlockSpec((1, 16), lambda i: (0, 0)),
    ],
    out_specs=pl.BlockSpec((BLK, 16), lambda i: (i, 0)),
    out_shape=jax.ShapeDtypeStruct((N, 16), jnp.float32),
)


def kernel(x, edge_index, W1, b1, W2, b2):
    src = edge_index[0].astype(jnp.int32)
    dst = edge_index[1].astype(jnp.int32)
    pad_src = jnp.arange(E_PAD - E, dtype=jnp.int32) % 2048
    pad_dst = N + jnp.arange(E_PAD - E, dtype=jnp.int32) % (N_PAD - N)
    src_rows = jnp.concatenate([src, pad_src]).reshape(ROWS, LANES)
    dst_rows = jnp.concatenate([dst, pad_dst]).reshape(ROWS, LANES)
    degs = _deg_call(dst_rows).reshape(NC, N_PAD)
    deg = (degs[0, :N] + degs[1, :N]).reshape(N, 1)

    h1, g0, g1 = _prep1(x, W1, deg)
    acc1 = _pass_b(g0, g1, src_rows, dst_rows)
    acc1 = acc1.reshape(NC, N_PAD, 16)
    h2, g2 = _mid(acc1, h1, deg, b1.reshape(1, 32), W2)
    acc2 = _pass_c(g2, src_rows, dst_rows)
    acc2 = acc2.reshape(NC, N_PAD, 16)
    return _fin(acc2, h2, deg, b2.reshape(1, 16))


# speedup vs baseline: 67.3875x; 1.1074x over previous
import functools

import jax
import jax.numpy as jnp
from jax import lax
from jax.experimental import pallas as pl
from jax.experimental.pallas import tpu as pltpu
from jax.experimental.pallas import tpu_sc as plsc

N = 100000
E = 3200000
NC, NS = 2, 16
LANES = 128
E_PAD = 3276800
ROWS = E_PAD // LANES
RPT = 6400
SUB = 800
N_PAD = NS * RPT
SENT = N_PAD - 1
CHUNK = 4
SUPER = 40
WSUB = 400
BLK = 4000
GRID = N // BLK

_mesh = plsc.VectorSubcoreMesh(core_axis_name="c", subcore_axis_name="s")
_sc_params = pltpu.CompilerParams(use_tc_tiling_on_sc=False)


def _deg_body(dst_rows, out_hbm, idx_v, ones_v, bounce, deg_sh, sem):
    cid = lax.axis_index("c")
    sid = lax.axis_index("s")

    def initv(i, carry):
        ones_v[pl.ds(i * 16, 16)] = jnp.ones((16,), jnp.float32)
        return carry

    lax.fori_loop(0, LANES // 16, initv, 0)

    def initb(i, carry):
        bounce[pl.ds(i * 16, 16)] = jnp.zeros((16,), jnp.float32)
        return carry

    lax.fori_loop(0, SUB // 16, initb, 0)
    for k in range(RPT // SUB):
        pltpu.sync_copy(bounce, deg_sh.at[pl.ds(sid * RPT + k * SUB, SUB)])
    plsc.subcore_barrier()
    rows_per_tile = ROWS // (NC * NS)
    base = (cid * NS + sid) * rows_per_tile

    def superchunk(i, carry):
        r0 = base + i * SUPER
        pltpu.sync_copy(dst_rows.at[pl.ds(r0, SUPER)], idx_v)
        handles = [
            pltpu.async_copy(ones_v, deg_sh.at[idx_v.at[j]], sem, add=True)
            for j in range(SUPER)
        ]
        for h in handles:
            h.wait()
        return carry

    lax.fori_loop(0, rows_per_tile // SUPER, superchunk, 0)
    plsc.subcore_barrier()
    for k in range(RPT // SUB):
        pltpu.sync_copy(deg_sh.at[pl.ds(sid * RPT + k * SUB, SUB)], bounce)
        pltpu.sync_copy(
            bounce,
            out_hbm.at[pl.ds(cid * N_PAD + sid * RPT + k * SUB, SUB)])


_deg_call = pl.kernel(
    _deg_body,
    out_type=jax.ShapeDtypeStruct((NC * N_PAD,), jnp.float32),
    mesh=_mesh,
    compiler_params=_sc_params,
    scratch_types=[
        pltpu.VMEM((SUPER, LANES), jnp.int32),
        pltpu.VMEM((LANES,), jnp.float32),
        pltpu.VMEM((SUB,), jnp.float32),
        pltpu.VMEM_SHARED((N_PAD,), jnp.float32),
        pltpu.SemaphoreType.DMA,
    ],
)


def _edge_pass_body(dual, *refs):
    if dual:
        (t0, t1, src_rows, dst_rows, out_hbm, sidx, didx, rows_a, rows_b,
         acc_sh, sem_a, sem_b, sem_sa, sem_sb) = refs
    else:
        (t0, src_rows, dst_rows, out_hbm, sidx, didx, rows_a, rows_b,
         acc_sh, sem_a, sem_b, sem_sa, sem_sb) = refs
        t1 = t0
    ssems = [sem_sa, sem_sb]
    cid = lax.axis_index("c")
    sid = lax.axis_index("s")

    def initz(i, carry):
        rows_a[i] = jnp.zeros((16,), jnp.float32)
        return carry

    lax.fori_loop(0, WSUB, initz, 0)
    for k in range(RPT // WSUB):
        pltpu.sync_copy(rows_a.at[pl.ds(0, WSUB)],
                        acc_sh.at[pl.ds(sid * RPT + k * WSUB, WSUB)])
    plsc.subcore_barrier()
    if dual:
        rows_per_tile = ROWS // NS
        base = sid * rows_per_tile
    else:
        rows_per_tile = ROWS // (NC * NS)
        base = (cid * NS + sid) * rows_per_tile

    nchunk = SUPER // CHUNK
    bufs = [(rows_a, sem_a), (rows_b, sem_b)]

    def superchunk(i, carry):
        s0 = base + i * SUPER
        pltpu.sync_copy(src_rows.at[pl.ds(s0, SUPER)], sidx)
        pltpu.sync_copy(dst_rows.at[pl.ds(s0, SUPER)], didx)

        def issue(c, tbl):
            buf, sem = bufs[c % 2]
            return [
                pltpu.async_copy(tbl.at[sidx.at[c * CHUNK + j]],
                                 buf.at[pl.ds(j * LANES, LANES)], sem)
                for j in range(CHUNK)
            ]

        def run(tbl):
            pending = issue(0, tbl)
            scat = {}
            for c in range(nchunk):
                if c + 1 < nchunk:
                    for h in scat.pop(c - 1, ()):
                        h.wait()
                    nxt = issue(c + 1, tbl)
                else:
                    nxt = []
                for h in pending:
                    h.wait()
                buf, _ = bufs[c % 2]
                ssem = ssems[c % 2]
                scat[c] = [
                    pltpu.async_copy(buf.at[pl.ds(j * LANES, LANES)],
                                     acc_sh.at[didx.at[c * CHUNK + j]],
                                     ssem, add=True)
                    for j in range(CHUNK)
                ]
                pending = nxt
            for hs in scat.values():
                for h in hs:
                    h.wait()

        if dual:
            @pl.when(cid == 0)
            def _():
                run(t0)

            @pl.when(cid == 1)
            def _():
                run(t1)
        else:
            run(t0)
        return carry

    lax.fori_loop(0, rows_per_tile // SUPER, superchunk, 0)
    plsc.subcore_barrier()
    for k in range(RPT // WSUB):
        pltpu.sync_copy(acc_sh.at[pl.ds(sid * RPT + k * WSUB, WSUB)],
                        rows_a.at[pl.ds(0, WSUB)])
        pltpu.sync_copy(
            rows_a.at[pl.ds(0, WSUB)],
            out_hbm.at[pl.ds(cid * N_PAD + sid * RPT + k * WSUB, WSUB)])


def _make_edge_pass(dual):
    return pl.kernel(
        functools.partial(_edge_pass_body, dual),
        out_type=jax.ShapeDtypeStruct((NC * N_PAD, 16), jnp.float32),
        mesh=_mesh,
        compiler_params=_sc_params,
        scratch_types=[
            pltpu.VMEM((SUPER, LANES), jnp.int32),
            pltpu.VMEM((SUPER, LANES), jnp.int32),
            pltpu.VMEM((CHUNK * LANES, 16), jnp.float32),
            pltpu.VMEM((CHUNK * LANES, 16), jnp.float32),
            pltpu.VMEM_SHARED((N_PAD, 16), jnp.float32),
            pltpu.SemaphoreType.DMA,
            pltpu.SemaphoreType.DMA,
            pltpu.SemaphoreType.DMA,
            pltpu.SemaphoreType.DMA,
        ],
    )


_pass_b = _make_edge_pass(True)
_pass_c = _make_edge_pass(False)


def _prep1_body(x_ref, w_ref, deg_ref, h_ref, g0_ref, g1_ref):
    dis = lax.rsqrt(deg_ref[...] + 1.0)
    h = jnp.dot(x_ref[...], w_ref[...], preferred_element_type=jnp.float32)
    h_ref[...] = h
    g = h * dis
    g0_ref[...] = g[:, :16]
    g1_ref[...] = g[:, 16:]


_prep1 = pl.pallas_call(
    _prep1_body,
    grid=(GRID,),
    in_specs=[
        pl.BlockSpec((BLK, 16), lambda i: (i, 0)),
        pl.BlockSpec((16, 32), lambda i: (0, 0)),
        pl.BlockSpec((BLK, 1), lambda i: (i, 0)),
    ],
    out_specs=[
        pl.BlockSpec((BLK, 32), lambda i: (i, 0)),
        pl.BlockSpec((BLK, 16), lambda i: (i, 0)),
        pl.BlockSpec((BLK, 16), lambda i: (i, 0)),
    ],
    out_shape=[
        jax.ShapeDtypeStruct((N_PAD, 32), jnp.float32),
        jax.ShapeDtypeStruct((N_PAD, 16), jnp.float32),
        jax.ShapeDtypeStruct((N_PAD, 16), jnp.float32),
    ],
)


def _mid_body(acc_ref, h1_ref, deg_ref, b1_ref, w2_ref, h2_ref, g2_ref):
    dis = lax.rsqrt(deg_ref[...] + 1.0)
    accc = jnp.concatenate([acc_ref[0], acc_ref[1]], axis=1)
    out1 = jnp.maximum(
        dis * accc + (dis * dis) * h1_ref[...] + b1_ref[...], 0.0)
    h2 = jnp.dot(out1, w2_ref[...], preferred_element_type=jnp.float32)
    h2_ref[...] = h2
    g2_ref[...] = h2 * dis


_mid = pl.pallas_call(
    _mid_body,
    grid=(GRID,),
    in_specs=[
        pl.BlockSpec((2, BLK, 16), lambda i: (0, i, 0)),
        pl.BlockSpec((BLK, 32), lambda i: (i, 0)),
        pl.BlockSpec((BLK, 1), lambda i: (i, 0)),
        pl.BlockSpec((1, 32), lambda i: (0, 0)),
        pl.BlockSpec((32, 16), lambda i: (0, 0)),
    ],
    out_specs=[
        pl.BlockSpec((BLK, 16), lambda i: (i, 0)),
        pl.BlockSpec((BLK, 16), lambda i: (i, 0)),
    ],
    out_shape=[
        jax.ShapeDtypeStruct((N_PAD, 16), jnp.float32),
        jax.ShapeDtypeStruct((N_PAD, 16), jnp.float32),
    ],
)


def _fin_body(acc_ref, h2_ref, deg_ref, b2_ref, out_ref):
    dis = lax.rsqrt(deg_ref[...] + 1.0)
    s = acc_ref[0] + acc_ref[1]
    out_ref[...] = dis * s + (dis * dis) * h2_ref[...] + b2_ref[...]


_fin = pl.pallas_call(
    _fin_body,
    grid=(GRID,),
    in_specs=[
        pl.BlockSpec((2, BLK, 16), lambda i: (0, i, 0)),
        pl.BlockSpec((BLK, 16), lambda i: (i, 0)),
        pl.BlockSpec((BLK, 1), lambda i: (i, 0)),
        pl.BlockSpec((1, 16), lambda i: (0, 0)),
    ],
    out_specs=pl.BlockSpec((BLK, 16), lambda i: (i, 0)),
    out_shape=jax.ShapeDtypeStruct((N, 16), jnp.float32),
)


def kernel(x, edge_index, W1, b1, W2, b2):
    src = edge_index[0].astype(jnp.int32)
    dst = edge_index[1].astype(jnp.int32)
    pad_src = jnp.arange(E_PAD - E, dtype=jnp.int32) % 2048
    pad_dst = N + jnp.arange(E_PAD - E, dtype=jnp.int32) % (N_PAD - N)
    src_rows = jnp.concatenate([src, pad_src]).reshape(ROWS, LANES)
    dst_rows = jnp.concatenate([dst, pad_dst]).reshape(ROWS, LANES)
    degs = _deg_call(dst_rows).reshape(NC, N_PAD)
    deg = (degs[0, :N] + degs[1, :N]).reshape(N, 1)

    h1, g0, g1 = _prep1(x, W1, deg)
    acc1 = _pass_b(g0, g1, src_rows, dst_rows)
    acc1 = acc1.reshape(NC, N_PAD, 16)
    h2, g2 = _mid(acc1, h1, deg, b1.reshape(1, 32), W2)
    acc2 = _pass_c(g2, src_rows, dst_rows)
    acc2 = acc2.reshape(NC, N_PAD, 16)
    return _fin(acc2, h2, deg, b2.reshape(1, 16))
